# baseline (speedup 1.0000x reference)
import jax
import jax.numpy as jnp
from jax import lax
from jax.experimental import pallas as pl
from jax.experimental.pallas import tpu as pltpu
from jax.experimental.pallas import tpu_sc as plsc

N = 16384
B = 64
BH = B // 2
NC = 2
NT = 16
LANES = 16

STREAM = 128
NSTREAM = 5
CH = STREAM * NSTREAM
NBUF = 4
CHUNKS = 28
KT = CH * CHUNKS
K_TOTAL = KT * NT
PACK = 3 * NSTREAM
ROWS_PER_TILE = N // NT


def _sc_body(x_hbm, pack_hbm, zeros_hbm, out_hbm,
             ib0, ib1, ib2, ib3, gb0, gb1, gb2, gb3, acc,
             is0, is1, is2, is3, gs0, gs1, gs2, gs3,
             ss0, ss1, ss2, ss3):
    c = lax.axis_index("c")
    s = lax.axis_index("s")
    ibufs = [ib0, ib1, ib2, ib3]
    isems = [is0, is1, is2, is3]
    gbufs = [gb0, gb1, gb2, gb3]
    gsems = [gs0, gs1, gs2, gs3]
    ssems = [ss0, ss1, ss2, ss3]

    pltpu.sync_copy(zeros_hbm.at[pl.ds(s * ROWS_PER_TILE, ROWS_PER_TILE)],
                    acc.at[pl.ds(s * ROWS_PER_TILE, ROWS_PER_TILE)])
    plsc.subcore_barrier()

    def fire_idx(chunk, b):
        pltpu.async_copy(pack_hbm.at[c, s, chunk], ibufs[b], isems[b])

    def drain_idx(b):
        pltpu.make_async_copy(pack_hbm.at[0, 0, 0], ibufs[b],
                              isems[b]).wait()

    def fire_gather(chunk_ib, b):
        for j in range(NSTREAM):
            pltpu.async_copy(
                x_hbm.at[ibufs[chunk_ib].at[j]],
                gbufs[b].at[pl.ds(j * STREAM, STREAM)], gsems[b])

    def drain(sem, b):
        pltpu.make_async_copy(x_hbm.at[pl.ds(0, CH)], gbufs[b], sem).wait()

    def compute(b):
        gb = gbufs[b]
        ib = ibufs[b]
        for j in range(NSTREAM):

            def grp(gg, carry, j=j, gb=gb, ib=ib):
                w16 = plsc.bitcast(
                    ib[2 * NSTREAM + j, pl.ds(gg * LANES, LANES)],
                    jnp.float32)
                base = j * STREAM + gg * LANES
                for k in range(LANES):
                    wk = w16[k]
                    r = base + k
                    gb[r, pl.ds(0, LANES)] = gb[r, pl.ds(0, LANES)] * wk
                    gb[r, pl.ds(LANES, LANES)] = (
                        gb[r, pl.ds(LANES, LANES)] * wk)
                return carry

            lax.fori_loop(0, STREAM // LANES, grp, 0)

    def fire_scatter(b):
        for j in range(NSTREAM):
            pltpu.async_copy(
                gbufs[b].at[pl.ds(j * STREAM, STREAM)],
                acc.at[ibufs[b].at[NSTREAM + j]], ssems[b], add=True)

    pltpu.sync_copy(pack_hbm.at[c, s, 0], ibufs[0])
    pltpu.sync_copy(pack_hbm.at[c, s, 1], ibufs[1])
    fire_idx(2, 2)
    fire_gather(0, 0)
    fire_gather(1, 1)

    def outer(c0, carry):
        for b in range(NBUF):
            ch = c0 + b
            drain(gsems[b], b)
            compute(b)
            if b == 0:
                @pl.when(c0 > 0)
                def _():
                    drain(ssems[NBUF - 1], NBUF - 1)
            else:
                drain(ssems[b - 1], b - 1)
            fire_scatter(b)
            g2 = (b + 2) % NBUF
            drain_idx(g2)
            fire_gather(g2, g2)
            i3 = (b + 3) % NBUF
            fire_idx(jnp.minimum(ch + 3, CHUNKS - 1), i3)
        return carry

    lax.fori_loop(0, CHUNKS // NBUF, lambda i, cr: outer(i * NBUF, cr), 0)

    drain(gsems[0], 0)
    drain(gsems[1], 1)
    drain_idx(2)
    drain(ssems[NBUF - 1], NBUF - 1)

    plsc.subcore_barrier()
    pltpu.sync_copy(acc.at[pl.ds(s * ROWS_PER_TILE, ROWS_PER_TILE)],
                    out_hbm.at[c, pl.ds(s * ROWS_PER_TILE, ROWS_PER_TILE)])


@jax.jit
def _sparse_matmul(xstack, pack, zeros):
    mesh = plsc.VectorSubcoreMesh(core_axis_name="c", subcore_axis_name="s",
                                  num_cores=NC, num_subcores=NT)
    run = pl.kernel(
        _sc_body,
        out_type=jax.ShapeDtypeStruct((NC, N, BH), jnp.float32),
        mesh=mesh,
        scratch_types=(
            [pltpu.VMEM((PACK, STREAM), jnp.int32) for _ in range(NBUF)]
            + [pltpu.VMEM((CH, BH), jnp.float32) for _ in range(NBUF)]
            + [pltpu.VMEM_SHARED((N, BH), jnp.float32)]
            + [pltpu.SemaphoreType.DMA for _ in range(3 * NBUF)]
        ),
        compiler_params=pltpu.CompilerParams(use_tc_tiling_on_sc=False,
                                             needs_layout_passes=False),
    )
    return run(xstack, pack, zeros)


def kernel(inputs, w, indices):
    nnz = indices.shape[0]
    rows = indices[:, 0].astype(jnp.int32)
    cols = indices[:, 1].astype(jnp.int32)

    pad = K_TOTAL - nnz
    rows = jnp.pad(rows, (0, pad))
    cols = jnp.pad(cols, (0, pad))
    wp = jnp.pad(w.astype(jnp.float32), (0, pad))
    w_i = lax.bitcast_convert_type(wp, jnp.int32)

    xT = inputs.astype(jnp.float32).T
    xstack = jnp.concatenate([xT[:, :BH], xT[:, BH:]], axis=0)

    def chunked(a):
        return a.reshape(NT, CHUNKS, NSTREAM, STREAM)

    cols_c = chunked(cols)
    w_c = chunked(w_i)
    packs = []
    for ci in range(NC):
        rows_c = chunked(rows + ci * N)
        packs.append(jnp.concatenate([rows_c, cols_c, w_c], axis=2))
    pack = jnp.stack(packs)

    zeros = jnp.zeros((N, BH), jnp.float32)

    o = _sparse_matmul(xstack, pack, zeros)
    return jnp.concatenate([o[0], o[1]], axis=1).T
